# baseline (speedup 1.0000x reference)
import jax
import jax.numpy as jnp
from jax.experimental import pallas as pl
from jax.experimental.pallas import tpu as pltpu

_VMEM_LIMIT = 100 * 1024 * 1024

_INV_2PI = 0.15915494309189535
_TWO_PI_HI = 6.2831854820251465
_TWO_PI_LO = -1.7484556025237907e-07


def _reduce_2pi(x):
    k = jnp.round(x * _INV_2PI)
    return x - k * _TWO_PI_HI - k * _TWO_PI_LO


def _sin_r(r):
    r2 = r * r
    p = jnp.float32(-2.036677351768823e-08)
    p = p * r2 + jnp.float32(2.6998364210557846e-06)
    p = p * r2 + jnp.float32(-0.00019808752397799424)
    p = p * r2 + jnp.float32(0.008332408078947556)
    p = p * r2 + jnp.float32(-0.16666553523387312)
    p = p * r2 + jnp.float32(0.999999604255913)
    return r * p


def _cos_r(r):
    r2 = r * r
    p = jnp.float32(-2.197962419847599e-07)
    p = p * r2 + jnp.float32(2.42045689199874e-05)
    p = p * r2 + jnp.float32(-0.001385892906818561)
    p = p * r2 + jnp.float32(0.04165982634184573)
    p = p * r2 + jnp.float32(-0.4999942726023237)
    p = p * r2 + jnp.float32(0.9999992223324515)
    return p


def _fast_sin(x):
    return _sin_r(_reduce_2pi(x))


def _fast_sin7(x):
    k = jnp.round(x * _INV_2PI)
    r = x - k * _TWO_PI_HI
    r2 = r * r
    p = jnp.float32(-0.00014508522994480244)
    p = p * r2 + jnp.float32(0.007958185693424477)
    p = p * r2 + jnp.float32(-0.1656675040609659)
    p = p * r2 + jnp.float32(0.9992763922392304)
    return r * p


def _gcn_l1_kernel(a_ref, dc_ref, dr_ref, xw1_ref, b1_ref, w2_ref, q_ref, *,
                   tk):
    n = a_ref.shape[1]
    acc = None
    for k0 in range(0, n, tk):
        ah = (a_ref[:, k0 : k0 + tk] * dc_ref[...]
              * dr_ref[:, k0 : k0 + tk])
        d = jnp.dot(ah, xw1_ref[k0 : k0 + tk, :],
                    preferred_element_type=jnp.float32)
        acc = d if acc is None else acc + d
    hmat = jnp.maximum(acc + b1_ref[...], 0.0)
    q_ref[...] = jnp.dot(hmat, w2_ref[...], preferred_element_type=jnp.float32)


def _gcn_l2_kernel(a_ref, dc_ref, dr_ref, q_ref, b2_ref, post_ref, *, tk):
    n = a_ref.shape[1]
    acc = None
    for k0 in range(0, n, tk):
        ah = (a_ref[:, k0 : k0 + tk] * dc_ref[...]
              * dr_ref[:, k0 : k0 + tk])
        d = jnp.dot(ah, q_ref[k0 : k0 + tk, :],
                    preferred_element_type=jnp.float32)
        acc = d if acc is None else acc + d
    post_ref[...] = acc + b2_ref[...]


def _gcn_forward(a, dinv_col, dinv_row, xw1, b1, w2, b2, *, bm, tk):
    n = a.shape[0]
    h = xw1.shape[1]
    import functools
    cparams = pltpu.CompilerParams(
        dimension_semantics=("parallel",), vmem_limit_bytes=_VMEM_LIMIT
    )
    q = pl.pallas_call(
        functools.partial(_gcn_l1_kernel, tk=tk),
        out_shape=jax.ShapeDtypeStruct((n, 1), jnp.float32),
        grid=(n // bm,),
        in_specs=[
            pl.BlockSpec((bm, n), lambda i: (i, 0)),
            pl.BlockSpec((bm, 1), lambda i: (i, 0)),
            pl.BlockSpec((1, n), lambda i: (0, 0)),
            pl.BlockSpec((n, h), lambda i: (0, 0)),
            pl.BlockSpec((1, h), lambda i: (0, 0)),
            pl.BlockSpec((h, 1), lambda i: (0, 0)),
        ],
        out_specs=pl.BlockSpec((bm, 1), lambda i: (i, 0)),
        compiler_params=cparams,
    )(a, dinv_col, dinv_row, xw1, b1, w2)

    post = pl.pallas_call(
        functools.partial(_gcn_l2_kernel, tk=tk),
        out_shape=jax.ShapeDtypeStruct((n, 1), jnp.float32),
        grid=(n // bm,),
        in_specs=[
            pl.BlockSpec((bm, n), lambda i: (i, 0)),
            pl.BlockSpec((bm, 1), lambda i: (i, 0)),
            pl.BlockSpec((1, n), lambda i: (0, 0)),
            pl.BlockSpec((n, 1), lambda i: (0, 0)),
            pl.BlockSpec((1, 1), lambda i: (0, 0)),
        ],
        out_specs=pl.BlockSpec((bm, 1), lambda i: (i, 0)),
        compiler_params=cparams,
    )(a, dinv_col, dinv_row, q, b2)
    return post


def _trig_kernel(zr_ref, b30_ref, c130_ref, g_ref):
    h = b30_ref.shape[0]
    arg = _reduce_2pi(b30_ref[...] * zr_ref[...] + c130_ref[...])
    g_ref[0:h, :] = _cos_r(arg)
    g_ref[h : 2 * h, :] = _sin_r(arg)


def _inr_kernel(z_ref, a30r_ref, v2t30_ref, c230_ref, v3_ref, c3_ref, g_ref,
                out_ref):
    ti = out_ref.shape[0]
    nj = out_ref.shape[1]
    tj = min(512, nj)
    v2t = v2t30_ref[...]
    c230 = c230_ref[...]
    v3c = v3_ref[...]
    c3 = c3_ref[...]
    a30r = a30r_ref[...]
    for ii in range(ti):
        p_row = _reduce_2pi(z_ref[ii : ii + 1, :] * a30r)
        w_cat = jnp.concatenate(
            [v2t * _sin_r(p_row), v2t * _cos_r(p_row)], axis=1
        )
        for j0 in range(0, nj, tj):
            m = (
                jnp.dot(w_cat, g_ref[:, j0 : j0 + tj],
                        preferred_element_type=jnp.float32)
                + c230
            )
            o = jnp.sum(_fast_sin7(m) * v3c, axis=0, keepdims=True) + c3
            out_ref[ii : ii + 1, j0 : j0 + tj] = o


def _inr_forward(post, v1, c1, v2, c2, v3, c3, *, ti):
    n = post.shape[0]
    h = v2.shape[0]

    z_row = jnp.transpose(post)
    a30r = 30.0 * v1[0:1, :]
    b30 = 30.0 * jnp.transpose(v1[1:2, :])
    c130 = 30.0 * jnp.transpose(c1)
    v2t30 = 30.0 * jnp.transpose(v2)
    c230 = 30.0 * jnp.transpose(c2)
    c3r = jnp.reshape(c3, (1, 1))

    bn = min(n, 512)
    g = pl.pallas_call(
        _trig_kernel,
        out_shape=jax.ShapeDtypeStruct((2 * h, n), jnp.float32),
        grid=(n // bn,),
        in_specs=[
            pl.BlockSpec((1, bn), lambda j: (0, j)),
            pl.BlockSpec((h, 1), lambda j: (0, 0)),
            pl.BlockSpec((h, 1), lambda j: (0, 0)),
        ],
        out_specs=pl.BlockSpec((2 * h, bn), lambda j: (0, j)),
        compiler_params=pltpu.CompilerParams(
            dimension_semantics=("parallel",), vmem_limit_bytes=_VMEM_LIMIT
        ),
    )(z_row, b30, c130)

    out2d = pl.pallas_call(
        _inr_kernel,
        out_shape=jax.ShapeDtypeStruct((n, n), jnp.float32),
        grid=(n // ti,),
        in_specs=[
            pl.BlockSpec((ti, 1), lambda i: (i, 0)),
            pl.BlockSpec((1, h), lambda i: (0, 0)),
            pl.BlockSpec((h, h), lambda i: (0, 0)),
            pl.BlockSpec((h, 1), lambda i: (0, 0)),
            pl.BlockSpec((h, 1), lambda i: (0, 0)),
            pl.BlockSpec((1, 1), lambda i: (0, 0)),
            pl.BlockSpec((2 * h, n), lambda i: (0, 0)),
        ],
        out_specs=pl.BlockSpec((ti, n), lambda i: (i, 0)),
        compiler_params=pltpu.CompilerParams(
            dimension_semantics=("parallel",), vmem_limit_bytes=_VMEM_LIMIT
        ),
    )(post, a30r, v2t30, c230, v3, c3r, g)

    return out2d.reshape(n * n, 1)


def kernel(x, edge_index, w1, b1, w2, b2, v1, c1, v2, c2, v3, c3):
    n = x.shape[0]

    a = jnp.zeros((n, n), jnp.float32)
    a = a.at[edge_index[0], edge_index[1]].set(1.0)
    a = a + jnp.eye(n, dtype=jnp.float32)
    dinv = 1.0 / jnp.sqrt(jnp.sum(a, axis=1))
    xw1 = jnp.dot(x, w1)

    post = _gcn_forward(
        a, dinv.reshape(n, 1), dinv.reshape(1, n), xw1, b1, w2, b2,
        bm=min(n, 512), tk=min(n, 1024),
    )
    out_inr = _inr_forward(post, v1, c1, v2, c2, v3, c3, ti=8 if n % 8 == 0 else n)
    return out_inr, post


# speedup vs baseline: 1.1302x; 1.1302x over previous
import jax
import jax.numpy as jnp
from jax.experimental import pallas as pl
from jax.experimental.pallas import tpu as pltpu

_VMEM_LIMIT = 100 * 1024 * 1024

_INV_2PI = 0.15915494309189535
_TWO_PI_HI = 6.2831854820251465
_TWO_PI_LO = -1.7484556025237907e-07


def _reduce_2pi(x):
    k = jnp.round(x * _INV_2PI)
    return x - k * _TWO_PI_HI - k * _TWO_PI_LO


def _sin_r(r):
    r2 = r * r
    p = jnp.float32(-2.036677351768823e-08)
    p = p * r2 + jnp.float32(2.6998364210557846e-06)
    p = p * r2 + jnp.float32(-0.00019808752397799424)
    p = p * r2 + jnp.float32(0.008332408078947556)
    p = p * r2 + jnp.float32(-0.16666553523387312)
    p = p * r2 + jnp.float32(0.999999604255913)
    return r * p


def _cos_r(r):
    r2 = r * r
    p = jnp.float32(-2.197962419847599e-07)
    p = p * r2 + jnp.float32(2.42045689199874e-05)
    p = p * r2 + jnp.float32(-0.001385892906818561)
    p = p * r2 + jnp.float32(0.04165982634184573)
    p = p * r2 + jnp.float32(-0.4999942726023237)
    p = p * r2 + jnp.float32(0.9999992223324515)
    return p


def _fast_sin(x):
    return _sin_r(_reduce_2pi(x))


def _sin_2pi_frac(s):
    t = s - jnp.round(s)
    t2 = t * t
    p = jnp.float32(-56.089591993946584)
    p = p * t2 + jnp.float32(77.93156727566378)
    p = p * t2 + jnp.float32(-41.09385974279256)
    p = p * t2 + jnp.float32(6.2786387455289)
    return t * p


def _gcn_l1_kernel(a_ref, dc_ref, dr_ref, xw1_ref, b1_ref, w2_ref, q_ref, *,
                   tk):
    n = a_ref.shape[1]
    acc = None
    for k0 in range(0, n, tk):
        ah = (a_ref[:, k0 : k0 + tk] * dc_ref[...]
              * dr_ref[:, k0 : k0 + tk])
        d = jnp.dot(ah, xw1_ref[k0 : k0 + tk, :],
                    preferred_element_type=jnp.float32)
        acc = d if acc is None else acc + d
    hmat = jnp.maximum(acc + b1_ref[...], 0.0)
    q_ref[...] = jnp.dot(hmat, w2_ref[...], preferred_element_type=jnp.float32)


def _gcn_l2_kernel(a_ref, dc_ref, dr_ref, q_ref, b2_ref, post_ref, *, tk):
    n = a_ref.shape[1]
    acc = None
    for k0 in range(0, n, tk):
        ah = (a_ref[:, k0 : k0 + tk] * dc_ref[...]
              * dr_ref[:, k0 : k0 + tk])
        d = jnp.dot(ah, q_ref[k0 : k0 + tk, :],
                    preferred_element_type=jnp.float32)
        acc = d if acc is None else acc + d
    post_ref[...] = acc + b2_ref[...]


def _gcn_forward(a, dinv_col, dinv_row, xw1, b1, w2, b2, *, bm, tk):
    n = a.shape[0]
    h = xw1.shape[1]
    import functools
    cparams = pltpu.CompilerParams(
        dimension_semantics=("parallel",), vmem_limit_bytes=_VMEM_LIMIT
    )
    q = pl.pallas_call(
        functools.partial(_gcn_l1_kernel, tk=tk),
        out_shape=jax.ShapeDtypeStruct((n, 1), jnp.float32),
        grid=(n // bm,),
        in_specs=[
            pl.BlockSpec((bm, n), lambda i: (i, 0)),
            pl.BlockSpec((bm, 1), lambda i: (i, 0)),
            pl.BlockSpec((1, n), lambda i: (0, 0)),
            pl.BlockSpec((n, h), lambda i: (0, 0)),
            pl.BlockSpec((1, h), lambda i: (0, 0)),
            pl.BlockSpec((h, 1), lambda i: (0, 0)),
        ],
        out_specs=pl.BlockSpec((bm, 1), lambda i: (i, 0)),
        compiler_params=cparams,
    )(a, dinv_col, dinv_row, xw1, b1, w2)

    post = pl.pallas_call(
        functools.partial(_gcn_l2_kernel, tk=tk),
        out_shape=jax.ShapeDtypeStruct((n, 1), jnp.float32),
        grid=(n // bm,),
        in_specs=[
            pl.BlockSpec((bm, n), lambda i: (i, 0)),
            pl.BlockSpec((bm, 1), lambda i: (i, 0)),
            pl.BlockSpec((1, n), lambda i: (0, 0)),
            pl.BlockSpec((n, 1), lambda i: (0, 0)),
            pl.BlockSpec((1, 1), lambda i: (0, 0)),
        ],
        out_specs=pl.BlockSpec((bm, 1), lambda i: (i, 0)),
        compiler_params=cparams,
    )(a, dinv_col, dinv_row, q, b2)
    return post


def _trig_kernel(zr_ref, b30_ref, c130_ref, g_ref):
    h = b30_ref.shape[0]
    arg = _reduce_2pi(b30_ref[...] * zr_ref[...] + c130_ref[...])
    g_ref[0:h, :] = _cos_r(arg)
    g_ref[h : 2 * h, :] = _sin_r(arg)


def _inr_kernel(z_ref, a30r_ref, v2t30_ref, c230_ref, v3_ref, c3_ref, g_ref,
                out_ref):
    ti = out_ref.shape[0]
    nj = out_ref.shape[1]
    tj = min(512, nj)
    v2t = v2t30_ref[...]
    c230 = c230_ref[...]
    v3c = v3_ref[...]
    c3 = c3_ref[...]
    a30r = a30r_ref[...]
    for ii in range(ti):
        p_row = _reduce_2pi(z_ref[ii : ii + 1, :] * a30r)
        w_cat = jnp.concatenate(
            [v2t * _sin_r(p_row), v2t * _cos_r(p_row)], axis=1
        )
        for j0 in range(0, nj, tj):
            m = (
                jnp.dot(w_cat, g_ref[:, j0 : j0 + tj],
                        preferred_element_type=jnp.float32)
                + c230
            )
            o = jnp.sum(_sin_2pi_frac(m) * v3c, axis=0, keepdims=True) + c3
            out_ref[ii : ii + 1, j0 : j0 + tj] = o


def _inr_forward(post, v1, c1, v2, c2, v3, c3, *, ti):
    n = post.shape[0]
    h = v2.shape[0]

    z_row = jnp.transpose(post)
    a30r = 30.0 * v1[0:1, :]
    b30 = 30.0 * jnp.transpose(v1[1:2, :])
    c130 = 30.0 * jnp.transpose(c1)
    v2t30 = (30.0 * _INV_2PI) * jnp.transpose(v2)
    c230 = (30.0 * _INV_2PI) * jnp.transpose(c2)
    c3r = jnp.reshape(c3, (1, 1))

    bn = min(n, 512)
    g = pl.pallas_call(
        _trig_kernel,
        out_shape=jax.ShapeDtypeStruct((2 * h, n), jnp.float32),
        grid=(n // bn,),
        in_specs=[
            pl.BlockSpec((1, bn), lambda j: (0, j)),
            pl.BlockSpec((h, 1), lambda j: (0, 0)),
            pl.BlockSpec((h, 1), lambda j: (0, 0)),
        ],
        out_specs=pl.BlockSpec((2 * h, bn), lambda j: (0, j)),
        compiler_params=pltpu.CompilerParams(
            dimension_semantics=("parallel",), vmem_limit_bytes=_VMEM_LIMIT
        ),
    )(z_row, b30, c130)

    out2d = pl.pallas_call(
        _inr_kernel,
        out_shape=jax.ShapeDtypeStruct((n, n), jnp.float32),
        grid=(n // ti,),
        in_specs=[
            pl.BlockSpec((ti, 1), lambda i: (i, 0)),
            pl.BlockSpec((1, h), lambda i: (0, 0)),
            pl.BlockSpec((h, h), lambda i: (0, 0)),
            pl.BlockSpec((h, 1), lambda i: (0, 0)),
            pl.BlockSpec((h, 1), lambda i: (0, 0)),
            pl.BlockSpec((1, 1), lambda i: (0, 0)),
            pl.BlockSpec((2 * h, n), lambda i: (0, 0)),
        ],
        out_specs=pl.BlockSpec((ti, n), lambda i: (i, 0)),
        compiler_params=pltpu.CompilerParams(
            dimension_semantics=("parallel",), vmem_limit_bytes=_VMEM_LIMIT
        ),
    )(post, a30r, v2t30, c230, v3, c3r, g)

    return out2d.reshape(n * n, 1)


def kernel(x, edge_index, w1, b1, w2, b2, v1, c1, v2, c2, v3, c3):
    n = x.shape[0]

    a = jnp.zeros((n, n), jnp.float32)
    a = a.at[edge_index[0], edge_index[1]].set(1.0)
    a = a + jnp.eye(n, dtype=jnp.float32)
    dinv = 1.0 / jnp.sqrt(jnp.sum(a, axis=1))
    xw1 = jnp.dot(x, w1)

    post = _gcn_forward(
        a, dinv.reshape(n, 1), dinv.reshape(1, n), xw1, b1, w2, b2,
        bm=min(n, 512), tk=min(n, 1024),
    )
    out_inr = _inr_forward(post, v1, c1, v2, c2, v3, c3, ti=8 if n % 8 == 0 else n)
    return out_inr, post


# speedup vs baseline: 1.1454x; 1.0135x over previous
import jax
import jax.numpy as jnp
from jax.experimental import pallas as pl
from jax.experimental.pallas import tpu as pltpu

_VMEM_LIMIT = 100 * 1024 * 1024

_INV_2PI = 0.15915494309189535
_TWO_PI_HI = 6.2831854820251465
_TWO_PI_LO = -1.7484556025237907e-07


def _reduce_2pi(x):
    k = jnp.round(x * _INV_2PI)
    return x - k * _TWO_PI_HI - k * _TWO_PI_LO


def _sin_r(r):
    r2 = r * r
    p = jnp.float32(-2.036677351768823e-08)
    p = p * r2 + jnp.float32(2.6998364210557846e-06)
    p = p * r2 + jnp.float32(-0.00019808752397799424)
    p = p * r2 + jnp.float32(0.008332408078947556)
    p = p * r2 + jnp.float32(-0.16666553523387312)
    p = p * r2 + jnp.float32(0.999999604255913)
    return r * p


def _cos_r(r):
    r2 = r * r
    p = jnp.float32(-2.197962419847599e-07)
    p = p * r2 + jnp.float32(2.42045689199874e-05)
    p = p * r2 + jnp.float32(-0.001385892906818561)
    p = p * r2 + jnp.float32(0.04165982634184573)
    p = p * r2 + jnp.float32(-0.4999942726023237)
    p = p * r2 + jnp.float32(0.9999992223324515)
    return p


def _fast_sin(x):
    return _sin_r(_reduce_2pi(x))


def _sin_2pi_frac(s):
    t = s - jnp.round(s)
    t2 = t * t
    p = jnp.float32(-56.089591993946584)
    p = p * t2 + jnp.float32(77.93156727566378)
    p = p * t2 + jnp.float32(-41.09385974279256)
    p = p * t2 + jnp.float32(6.2786387455289)
    return t * p


def _gcn_l1_kernel(a_ref, dc_ref, dr_ref, xw1_ref, b1_ref, w2_ref, q_ref, *,
                   tk):
    n = a_ref.shape[1]
    acc = None
    for k0 in range(0, n, tk):
        ah = (a_ref[:, k0 : k0 + tk] * dc_ref[...]
              * dr_ref[:, k0 : k0 + tk])
        d = jnp.dot(ah, xw1_ref[k0 : k0 + tk, :],
                    preferred_element_type=jnp.float32)
        acc = d if acc is None else acc + d
    hmat = jnp.maximum(acc + b1_ref[...], 0.0)
    q_ref[...] = jnp.dot(hmat, w2_ref[...], preferred_element_type=jnp.float32)


def _gcn_l2_kernel(a_ref, dc_ref, dr_ref, q_ref, b2_ref, post_ref, *, tk):
    n = a_ref.shape[1]
    acc = None
    for k0 in range(0, n, tk):
        ah = (a_ref[:, k0 : k0 + tk] * dc_ref[...]
              * dr_ref[:, k0 : k0 + tk])
        d = jnp.dot(ah, q_ref[k0 : k0 + tk, :],
                    preferred_element_type=jnp.float32)
        acc = d if acc is None else acc + d
    post_ref[...] = acc + b2_ref[...]


def _gcn_forward(a, dinv_col, dinv_row, xw1, b1, w2, b2, *, bm, tk):
    n = a.shape[0]
    h = xw1.shape[1]
    import functools
    cparams = pltpu.CompilerParams(
        dimension_semantics=("parallel",), vmem_limit_bytes=_VMEM_LIMIT
    )
    q = pl.pallas_call(
        functools.partial(_gcn_l1_kernel, tk=tk),
        out_shape=jax.ShapeDtypeStruct((n, 1), jnp.float32),
        grid=(n // bm,),
        in_specs=[
            pl.BlockSpec((bm, n), lambda i: (i, 0)),
            pl.BlockSpec((bm, 1), lambda i: (i, 0)),
            pl.BlockSpec((1, n), lambda i: (0, 0)),
            pl.BlockSpec((n, h), lambda i: (0, 0)),
            pl.BlockSpec((1, h), lambda i: (0, 0)),
            pl.BlockSpec((h, 1), lambda i: (0, 0)),
        ],
        out_specs=pl.BlockSpec((bm, 1), lambda i: (i, 0)),
        compiler_params=cparams,
    )(a, dinv_col, dinv_row, xw1, b1, w2)

    post = pl.pallas_call(
        functools.partial(_gcn_l2_kernel, tk=tk),
        out_shape=jax.ShapeDtypeStruct((n, 1), jnp.float32),
        grid=(n // bm,),
        in_specs=[
            pl.BlockSpec((bm, n), lambda i: (i, 0)),
            pl.BlockSpec((bm, 1), lambda i: (i, 0)),
            pl.BlockSpec((1, n), lambda i: (0, 0)),
            pl.BlockSpec((n, 1), lambda i: (0, 0)),
            pl.BlockSpec((1, 1), lambda i: (0, 0)),
        ],
        out_specs=pl.BlockSpec((bm, 1), lambda i: (i, 0)),
        compiler_params=cparams,
    )(a, dinv_col, dinv_row, q, b2)
    return post


def _trig_kernel(zr_ref, b30_ref, c130_ref, g_ref):
    h = b30_ref.shape[0]
    arg = _reduce_2pi(b30_ref[...] * zr_ref[...] + c130_ref[...])
    g_ref[0:h, :] = _cos_r(arg)
    g_ref[h : 2 * h, :] = _sin_r(arg)


def _inr_kernel(z_ref, a30r_ref, v2t30_ref, c230_ref, v3_ref, c3_ref, g_ref,
                out_ref):
    ti = out_ref.shape[0]
    nj = out_ref.shape[1]
    tj = min(512, nj)
    v2t = v2t30_ref[...]
    c230 = c230_ref[...]
    v3c = v3_ref[...]
    c3 = c3_ref[...]
    a30r = a30r_ref[...]
    for ii in range(ti):
        p_row = _reduce_2pi(z_ref[ii : ii + 1, :] * a30r)
        w_cat = jnp.concatenate(
            [v2t * _sin_r(p_row), v2t * _cos_r(p_row)], axis=1
        )
        for j0 in range(0, nj, tj):
            m = (
                jnp.dot(w_cat, g_ref[:, j0 : j0 + tj],
                        preferred_element_type=jnp.float32)
                + c230
            )
            o = jnp.sum(_sin_2pi_frac(m) * v3c, axis=0, keepdims=True) + c3
            out_ref[ii : ii + 1, j0 : j0 + tj] = o


def _inr_forward(post, v1, c1, v2, c2, v3, c3, *, ti):
    n = post.shape[0]
    h = v2.shape[0]

    z_row = jnp.transpose(post)
    a30r = 30.0 * v1[0:1, :]
    b30 = 30.0 * jnp.transpose(v1[1:2, :])
    c130 = 30.0 * jnp.transpose(c1)
    v2t30 = (30.0 * _INV_2PI) * jnp.transpose(v2)
    c230 = (30.0 * _INV_2PI) * jnp.transpose(c2)
    c3r = jnp.reshape(c3, (1, 1))

    bn = min(n, 512)
    g = pl.pallas_call(
        _trig_kernel,
        out_shape=jax.ShapeDtypeStruct((2 * h, n), jnp.float32),
        grid=(n // bn,),
        in_specs=[
            pl.BlockSpec((1, bn), lambda j: (0, j)),
            pl.BlockSpec((h, 1), lambda j: (0, 0)),
            pl.BlockSpec((h, 1), lambda j: (0, 0)),
        ],
        out_specs=pl.BlockSpec((2 * h, bn), lambda j: (0, j)),
        compiler_params=pltpu.CompilerParams(
            dimension_semantics=("parallel",), vmem_limit_bytes=_VMEM_LIMIT
        ),
    )(z_row, b30, c130)

    out2d = pl.pallas_call(
        _inr_kernel,
        out_shape=jax.ShapeDtypeStruct((n, n), jnp.float32),
        grid=(n // ti,),
        in_specs=[
            pl.BlockSpec((ti, 1), lambda i: (i, 0)),
            pl.BlockSpec((1, h), lambda i: (0, 0)),
            pl.BlockSpec((h, h), lambda i: (0, 0)),
            pl.BlockSpec((h, 1), lambda i: (0, 0)),
            pl.BlockSpec((h, 1), lambda i: (0, 0)),
            pl.BlockSpec((1, 1), lambda i: (0, 0)),
            pl.BlockSpec((2 * h, n), lambda i: (0, 0)),
        ],
        out_specs=pl.BlockSpec((ti, n), lambda i: (i, 0)),
        compiler_params=pltpu.CompilerParams(
            dimension_semantics=("parallel",), vmem_limit_bytes=_VMEM_LIMIT
        ),
    )(post, a30r, v2t30, c230, v3, c3r, g)

    return out2d.reshape(n * n, 1)


def kernel(x, edge_index, w1, b1, w2, b2, v1, c1, v2, c2, v3, c3):
    n = x.shape[0]

    a = jnp.zeros((n, n), jnp.float32)
    a = a.at[edge_index[0], edge_index[1]].set(1.0)
    a = a + jnp.eye(n, dtype=jnp.float32)
    dinv = 1.0 / jnp.sqrt(jnp.sum(a, axis=1))
    xw1 = jnp.dot(x, w1)

    post = _gcn_forward(
        a, dinv.reshape(n, 1), dinv.reshape(1, n), xw1, b1, w2, b2,
        bm=min(n, 512), tk=min(n, 1024),
    )
    out_inr = _inr_forward(post, v1, c1, v2, c2, v3, c3, ti=16 if n % 16 == 0 else n)
    return out_inr, post


# speedup vs baseline: 1.1790x; 1.0293x over previous
import jax
import jax.numpy as jnp
from jax.experimental import pallas as pl
from jax.experimental.pallas import tpu as pltpu

_VMEM_LIMIT = 100 * 1024 * 1024

_INV_2PI = 0.15915494309189535
_TWO_PI_HI = 6.2831854820251465
_TWO_PI_LO = -1.7484556025237907e-07


def _reduce_2pi(x):
    k = jnp.round(x * _INV_2PI)
    return x - k * _TWO_PI_HI - k * _TWO_PI_LO


def _sin_r(r):
    r2 = r * r
    p = jnp.float32(-2.036677351768823e-08)
    p = p * r2 + jnp.float32(2.6998364210557846e-06)
    p = p * r2 + jnp.float32(-0.00019808752397799424)
    p = p * r2 + jnp.float32(0.008332408078947556)
    p = p * r2 + jnp.float32(-0.16666553523387312)
    p = p * r2 + jnp.float32(0.999999604255913)
    return r * p


def _cos_r(r):
    r2 = r * r
    p = jnp.float32(-2.197962419847599e-07)
    p = p * r2 + jnp.float32(2.42045689199874e-05)
    p = p * r2 + jnp.float32(-0.001385892906818561)
    p = p * r2 + jnp.float32(0.04165982634184573)
    p = p * r2 + jnp.float32(-0.4999942726023237)
    p = p * r2 + jnp.float32(0.9999992223324515)
    return p


def _fast_sin(x):
    return _sin_r(_reduce_2pi(x))


def _sin_2pi_frac(s):
    t = s - jnp.round(s)
    t2 = t * t
    p = jnp.float32(-56.089591993946584)
    p = p * t2 + jnp.float32(77.93156727566378)
    p = p * t2 + jnp.float32(-41.09385974279256)
    p = p * t2 + jnp.float32(6.2786387455289)
    return t * p


def _gcn_l1_kernel(a_ref, dc_ref, dr_ref, xw1_ref, b1_ref, w2_ref, q_ref, *,
                   tk):
    n = a_ref.shape[1]
    acc = None
    for k0 in range(0, n, tk):
        ah = (a_ref[:, k0 : k0 + tk] * dc_ref[...]
              * dr_ref[:, k0 : k0 + tk])
        d = jnp.dot(ah, xw1_ref[k0 : k0 + tk, :],
                    preferred_element_type=jnp.float32)
        acc = d if acc is None else acc + d
    hmat = jnp.maximum(acc + b1_ref[...], 0.0)
    q_ref[...] = jnp.dot(hmat, w2_ref[...], preferred_element_type=jnp.float32)


def _gcn_l2_kernel(a_ref, dc_ref, dr_ref, q_ref, b2_ref, post_ref, *, tk):
    n = a_ref.shape[1]
    acc = None
    for k0 in range(0, n, tk):
        ah = (a_ref[:, k0 : k0 + tk] * dc_ref[...]
              * dr_ref[:, k0 : k0 + tk])
        d = jnp.dot(ah, q_ref[k0 : k0 + tk, :],
                    preferred_element_type=jnp.float32)
        acc = d if acc is None else acc + d
    post_ref[...] = acc + b2_ref[...]


def _gcn_forward(a, dinv_col, dinv_row, xw1, b1, w2, b2, *, bm, tk):
    n = a.shape[0]
    h = xw1.shape[1]
    import functools
    cparams = pltpu.CompilerParams(
        dimension_semantics=("parallel",), vmem_limit_bytes=_VMEM_LIMIT
    )
    q = pl.pallas_call(
        functools.partial(_gcn_l1_kernel, tk=tk),
        out_shape=jax.ShapeDtypeStruct((n, 1), jnp.float32),
        grid=(n // bm,),
        in_specs=[
            pl.BlockSpec((bm, n), lambda i: (i, 0)),
            pl.BlockSpec((bm, 1), lambda i: (i, 0)),
            pl.BlockSpec((1, n), lambda i: (0, 0)),
            pl.BlockSpec((n, h), lambda i: (0, 0)),
            pl.BlockSpec((1, h), lambda i: (0, 0)),
            pl.BlockSpec((h, 1), lambda i: (0, 0)),
        ],
        out_specs=pl.BlockSpec((bm, 1), lambda i: (i, 0)),
        compiler_params=cparams,
    )(a, dinv_col, dinv_row, xw1, b1, w2)

    post = pl.pallas_call(
        functools.partial(_gcn_l2_kernel, tk=tk),
        out_shape=jax.ShapeDtypeStruct((n, 1), jnp.float32),
        grid=(n // bm,),
        in_specs=[
            pl.BlockSpec((bm, n), lambda i: (i, 0)),
            pl.BlockSpec((bm, 1), lambda i: (i, 0)),
            pl.BlockSpec((1, n), lambda i: (0, 0)),
            pl.BlockSpec((n, 1), lambda i: (0, 0)),
            pl.BlockSpec((1, 1), lambda i: (0, 0)),
        ],
        out_specs=pl.BlockSpec((bm, 1), lambda i: (i, 0)),
        compiler_params=cparams,
    )(a, dinv_col, dinv_row, q, b2)
    return post


def _trig_kernel(zr_ref, b30_ref, c130_ref, g_ref):
    h = b30_ref.shape[0]
    arg = _reduce_2pi(b30_ref[...] * zr_ref[...] + c130_ref[...])
    g_ref[0:h, :] = _cos_r(arg)
    g_ref[h : 2 * h, :] = _sin_r(arg)


def _inr_kernel(z_ref, a30r_ref, v2t30_ref, c230_ref, qcv_ref, c3_ref, g_ref,
                out_ref):
    ti = out_ref.shape[0]
    nj = out_ref.shape[1]
    tj = min(512, nj)
    v2t = v2t30_ref[...]
    c230 = c230_ref[...]
    c3 = c3_ref[...]
    a30r = a30r_ref[...]
    qc1 = qcv_ref[:, 0:1]
    qc3 = qcv_ref[:, 1:2]
    qc5 = qcv_ref[:, 2:3]
    qc7 = qcv_ref[:, 3:4]
    for ii in range(ti):
        p_row = _reduce_2pi(z_ref[ii : ii + 1, :] * a30r)
        w_cat = jnp.concatenate(
            [v2t * _sin_r(p_row), v2t * _cos_r(p_row)], axis=1
        )
        for j0 in range(0, nj, tj):
            s = (
                jnp.dot(w_cat, g_ref[:, j0 : j0 + tj],
                        preferred_element_type=jnp.float32)
                + c230
            )
            t = s - jnp.round(s)
            t2 = t * t
            p = qc7
            p = p * t2 + qc5
            p = p * t2 + qc3
            p = p * t2 + qc1
            o = jnp.sum(t * p, axis=0, keepdims=True) + c3
            out_ref[ii : ii + 1, j0 : j0 + tj] = o


def _inr_forward(post, v1, c1, v2, c2, v3, c3, *, ti):
    n = post.shape[0]
    h = v2.shape[0]

    z_row = jnp.transpose(post)
    a30r = 30.0 * v1[0:1, :]
    b30 = 30.0 * jnp.transpose(v1[1:2, :])
    c130 = 30.0 * jnp.transpose(c1)
    v2t30 = (30.0 * _INV_2PI) * jnp.transpose(v2)
    c230 = (30.0 * _INV_2PI) * jnp.transpose(c2)
    c3r = jnp.reshape(c3, (1, 1))
    qcv = jnp.concatenate(
        [jnp.float32(6.2786387455289) * v3,
         jnp.float32(-41.09385974279256) * v3,
         jnp.float32(77.93156727566378) * v3,
         jnp.float32(-56.089591993946584) * v3], axis=1)

    bn = min(n, 512)
    g = pl.pallas_call(
        _trig_kernel,
        out_shape=jax.ShapeDtypeStruct((2 * h, n), jnp.float32),
        grid=(n // bn,),
        in_specs=[
            pl.BlockSpec((1, bn), lambda j: (0, j)),
            pl.BlockSpec((h, 1), lambda j: (0, 0)),
            pl.BlockSpec((h, 1), lambda j: (0, 0)),
        ],
        out_specs=pl.BlockSpec((2 * h, bn), lambda j: (0, j)),
        compiler_params=pltpu.CompilerParams(
            dimension_semantics=("parallel",), vmem_limit_bytes=_VMEM_LIMIT
        ),
    )(z_row, b30, c130)

    out2d = pl.pallas_call(
        _inr_kernel,
        out_shape=jax.ShapeDtypeStruct((n, n), jnp.float32),
        grid=(n // ti,),
        in_specs=[
            pl.BlockSpec((ti, 1), lambda i: (i, 0)),
            pl.BlockSpec((1, h), lambda i: (0, 0)),
            pl.BlockSpec((h, h), lambda i: (0, 0)),
            pl.BlockSpec((h, 1), lambda i: (0, 0)),
            pl.BlockSpec((h, 4), lambda i: (0, 0)),
            pl.BlockSpec((1, 1), lambda i: (0, 0)),
            pl.BlockSpec((2 * h, n), lambda i: (0, 0)),
        ],
        out_specs=pl.BlockSpec((ti, n), lambda i: (i, 0)),
        compiler_params=pltpu.CompilerParams(
            dimension_semantics=("parallel",), vmem_limit_bytes=_VMEM_LIMIT
        ),
    )(post, a30r, v2t30, c230, qcv, c3r, g)

    return out2d.reshape(n * n, 1)


def kernel(x, edge_index, w1, b1, w2, b2, v1, c1, v2, c2, v3, c3):
    n = x.shape[0]

    a = jnp.zeros((n, n), jnp.float32)
    a = a.at[edge_index[0], edge_index[1]].set(1.0)
    a = a + jnp.eye(n, dtype=jnp.float32)
    dinv = 1.0 / jnp.sqrt(jnp.sum(a, axis=1))
    xw1 = jnp.dot(x, w1)

    post = _gcn_forward(
        a, dinv.reshape(n, 1), dinv.reshape(1, n), xw1, b1, w2, b2,
        bm=min(n, 512), tk=min(n, 1024),
    )
    out_inr = _inr_forward(post, v1, c1, v2, c2, v3, c3, ti=16 if n % 16 == 0 else n)
    return out_inr, post


# speedup vs baseline: 1.2035x; 1.0208x over previous
import jax
import jax.numpy as jnp
from jax.experimental import pallas as pl
from jax.experimental.pallas import tpu as pltpu

_VMEM_LIMIT = 100 * 1024 * 1024

_INV_2PI = 0.15915494309189535
_TWO_PI_HI = 6.2831854820251465
_TWO_PI_LO = -1.7484556025237907e-07


def _reduce_2pi(x):
    k = jnp.round(x * _INV_2PI)
    return x - k * _TWO_PI_HI - k * _TWO_PI_LO


def _sin_r(r):
    r2 = r * r
    p = jnp.float32(-2.036677351768823e-08)
    p = p * r2 + jnp.float32(2.6998364210557846e-06)
    p = p * r2 + jnp.float32(-0.00019808752397799424)
    p = p * r2 + jnp.float32(0.008332408078947556)
    p = p * r2 + jnp.float32(-0.16666553523387312)
    p = p * r2 + jnp.float32(0.999999604255913)
    return r * p


def _cos_r(r):
    r2 = r * r
    p = jnp.float32(-2.197962419847599e-07)
    p = p * r2 + jnp.float32(2.42045689199874e-05)
    p = p * r2 + jnp.float32(-0.001385892906818561)
    p = p * r2 + jnp.float32(0.04165982634184573)
    p = p * r2 + jnp.float32(-0.4999942726023237)
    p = p * r2 + jnp.float32(0.9999992223324515)
    return p


def _fast_sin(x):
    return _sin_r(_reduce_2pi(x))


def _sin_2pi_frac(s):
    t = s - jnp.round(s)
    t2 = t * t
    p = jnp.float32(-56.089591993946584)
    p = p * t2 + jnp.float32(77.93156727566378)
    p = p * t2 + jnp.float32(-41.09385974279256)
    p = p * t2 + jnp.float32(6.2786387455289)
    return t * p


def _gcn_l1_kernel(a_ref, dc_ref, dr_ref, xw1_ref, b1_ref, w2_ref, q_ref, *,
                   tk):
    n = a_ref.shape[1]
    acc = None
    for k0 in range(0, n, tk):
        ah = (a_ref[:, k0 : k0 + tk] * dc_ref[...]
              * dr_ref[:, k0 : k0 + tk])
        d = jnp.dot(ah, xw1_ref[k0 : k0 + tk, :],
                    preferred_element_type=jnp.float32)
        acc = d if acc is None else acc + d
    hmat = jnp.maximum(acc + b1_ref[...], 0.0)
    q_ref[...] = jnp.dot(hmat, w2_ref[...], preferred_element_type=jnp.float32)


def _gcn_l2_kernel(a_ref, dc_ref, dr_ref, q_ref, b2_ref, post_ref, *, tk):
    n = a_ref.shape[1]
    acc = None
    for k0 in range(0, n, tk):
        ah = (a_ref[:, k0 : k0 + tk] * dc_ref[...]
              * dr_ref[:, k0 : k0 + tk])
        d = jnp.dot(ah, q_ref[k0 : k0 + tk, :],
                    preferred_element_type=jnp.float32)
        acc = d if acc is None else acc + d
    post_ref[...] = acc + b2_ref[...]


def _gcn_forward(a, dinv_col, dinv_row, xw1, b1, w2, b2, *, bm, tk):
    n = a.shape[0]
    h = xw1.shape[1]
    import functools
    cparams = pltpu.CompilerParams(
        dimension_semantics=("parallel",), vmem_limit_bytes=_VMEM_LIMIT
    )
    q = pl.pallas_call(
        functools.partial(_gcn_l1_kernel, tk=tk),
        out_shape=jax.ShapeDtypeStruct((n, 1), jnp.float32),
        grid=(n // bm,),
        in_specs=[
            pl.BlockSpec((bm, n), lambda i: (i, 0)),
            pl.BlockSpec((bm, 1), lambda i: (i, 0)),
            pl.BlockSpec((1, n), lambda i: (0, 0)),
            pl.BlockSpec((n, h), lambda i: (0, 0)),
            pl.BlockSpec((1, h), lambda i: (0, 0)),
            pl.BlockSpec((h, 1), lambda i: (0, 0)),
        ],
        out_specs=pl.BlockSpec((bm, 1), lambda i: (i, 0)),
        compiler_params=cparams,
    )(a, dinv_col, dinv_row, xw1, b1, w2)

    post = pl.pallas_call(
        functools.partial(_gcn_l2_kernel, tk=tk),
        out_shape=jax.ShapeDtypeStruct((n, 1), jnp.float32),
        grid=(n // bm,),
        in_specs=[
            pl.BlockSpec((bm, n), lambda i: (i, 0)),
            pl.BlockSpec((bm, 1), lambda i: (i, 0)),
            pl.BlockSpec((1, n), lambda i: (0, 0)),
            pl.BlockSpec((n, 1), lambda i: (0, 0)),
            pl.BlockSpec((1, 1), lambda i: (0, 0)),
        ],
        out_specs=pl.BlockSpec((bm, 1), lambda i: (i, 0)),
        compiler_params=cparams,
    )(a, dinv_col, dinv_row, q, b2)
    return post


def _trig_kernel(zr_ref, b30_ref, c130_ref, g_ref):
    h = b30_ref.shape[0]
    arg = _reduce_2pi(b30_ref[...] * zr_ref[...] + c130_ref[...])
    g_ref[0:h, :] = _cos_r(arg)
    g_ref[h : 2 * h, :] = _sin_r(arg)
    bn = g_ref.shape[1]
    g_ref[2 * h : 2 * h + 8, :] = jnp.concatenate(
        [jnp.ones((1, bn), jnp.float32), jnp.zeros((7, bn), jnp.float32)],
        axis=0)


def _inr_kernel(z_ref, a30r_ref, v2t30_ref, c230_ref, qcv_ref, c3_ref, g_ref,
                out_ref):
    ti = out_ref.shape[0]
    nj = out_ref.shape[1]
    tj = min(512, nj)
    v2t = v2t30_ref[...]
    c230 = c230_ref[...]
    c3 = c3_ref[...]
    a30r = a30r_ref[...]
    qc1 = qcv_ref[:, 0:1]
    qc3 = qcv_ref[:, 1:2]
    qc5 = qcv_ref[:, 2:3]
    qc7 = qcv_ref[:, 3:4]
    for ii in range(ti):
        p_row = _reduce_2pi(z_ref[ii : ii + 1, :] * a30r)
        w_cat = jnp.concatenate(
            [v2t * _sin_r(p_row), v2t * _cos_r(p_row), c230], axis=1
        )
        for j0 in range(0, nj, tj):
            s = jnp.dot(w_cat, g_ref[:, j0 : j0 + tj],
                        preferred_element_type=jnp.float32)
            t = s - jnp.round(s)
            t2 = t * t
            p = qc7
            p = p * t2 + qc5
            p = p * t2 + qc3
            p = p * t2 + qc1
            o = jnp.sum(t * p, axis=0, keepdims=True) + c3
            out_ref[ii : ii + 1, j0 : j0 + tj] = o


def _inr_forward(post, v1, c1, v2, c2, v3, c3, *, ti):
    n = post.shape[0]
    h = v2.shape[0]

    z_row = jnp.transpose(post)
    a30r = 30.0 * v1[0:1, :]
    b30 = 30.0 * jnp.transpose(v1[1:2, :])
    c130 = 30.0 * jnp.transpose(c1)
    v2t30 = (30.0 * _INV_2PI) * jnp.transpose(v2)
    c230 = jnp.concatenate(
        [(30.0 * _INV_2PI) * jnp.transpose(c2), jnp.zeros((h, 7), jnp.float32)],
        axis=1)
    c3r = jnp.reshape(c3, (1, 1))
    qcv = jnp.concatenate(
        [jnp.float32(6.2786387455289) * v3,
         jnp.float32(-41.09385974279256) * v3,
         jnp.float32(77.93156727566378) * v3,
         jnp.float32(-56.089591993946584) * v3], axis=1)

    bn = min(n, 512)
    g = pl.pallas_call(
        _trig_kernel,
        out_shape=jax.ShapeDtypeStruct((2 * h + 8, n), jnp.float32),
        grid=(n // bn,),
        in_specs=[
            pl.BlockSpec((1, bn), lambda j: (0, j)),
            pl.BlockSpec((h, 1), lambda j: (0, 0)),
            pl.BlockSpec((h, 1), lambda j: (0, 0)),
        ],
        out_specs=pl.BlockSpec((2 * h + 8, bn), lambda j: (0, j)),
        compiler_params=pltpu.CompilerParams(
            dimension_semantics=("parallel",), vmem_limit_bytes=_VMEM_LIMIT
        ),
    )(z_row, b30, c130)

    out2d = pl.pallas_call(
        _inr_kernel,
        out_shape=jax.ShapeDtypeStruct((n, n), jnp.float32),
        grid=(n // ti,),
        in_specs=[
            pl.BlockSpec((ti, 1), lambda i: (i, 0)),
            pl.BlockSpec((1, h), lambda i: (0, 0)),
            pl.BlockSpec((h, h), lambda i: (0, 0)),
            pl.BlockSpec((h, 8), lambda i: (0, 0)),
            pl.BlockSpec((h, 4), lambda i: (0, 0)),
            pl.BlockSpec((1, 1), lambda i: (0, 0)),
            pl.BlockSpec((2 * h + 8, n), lambda i: (0, 0)),
        ],
        out_specs=pl.BlockSpec((ti, n), lambda i: (i, 0)),
        compiler_params=pltpu.CompilerParams(
            dimension_semantics=("parallel",), vmem_limit_bytes=_VMEM_LIMIT
        ),
    )(post, a30r, v2t30, c230, qcv, c3r, g)

    return out2d.reshape(n * n, 1)


def kernel(x, edge_index, w1, b1, w2, b2, v1, c1, v2, c2, v3, c3):
    n = x.shape[0]

    a = jnp.zeros((n, n), jnp.float32)
    a = a.at[edge_index[0], edge_index[1]].set(1.0)
    a = a + jnp.eye(n, dtype=jnp.float32)
    dinv = 1.0 / jnp.sqrt(jnp.sum(a, axis=1))
    xw1 = jnp.dot(x, w1)

    post = _gcn_forward(
        a, dinv.reshape(n, 1), dinv.reshape(1, n), xw1, b1, w2, b2,
        bm=min(n, 512), tk=min(n, 1024),
    )
    out_inr = _inr_forward(post, v1, c1, v2, c2, v3, c3, ti=16 if n % 16 == 0 else n)
    return out_inr, post
